# baseline (speedup 1.0000x reference)
import functools

import jax
import jax.numpy as jnp
from jax import lax
from jax.experimental import pallas as pl
from jax.experimental.pallas import tpu as pltpu
from jax.experimental.pallas import tpu_sc as plsc

B, C, D, S = 32, 2048, 64, 4
NC, NS = 2, 16
NW = NC * NS
DBLK = 4
NBLK = D // DBLK
QV = C // 16
STEPS = C.bit_length()


@functools.cache
def _make_sc_kernel():
    @functools.partial(
        pl.kernel,
        out_type=(jax.ShapeDtypeStruct((B * S * D, C), jnp.float32),
                  jax.ShapeDtypeStruct((B * S * D, C), jnp.float32)),
        mesh=plsc.VectorSubcoreMesh(core_axis_name="c", subcore_axis_name="s"),
        compiler_params=pltpu.CompilerParams(use_tc_tiling_on_sc=False,
                                             needs_layout_passes=False),
        scratch_types=[
            pltpu.VMEM((C,), jnp.float32),
            pltpu.VMEM((S * C,), jnp.float32),
            pltpu.VMEM((S * C,), jnp.int32),
            pltpu.VMEM((2, DBLK, C), jnp.float32),
            pltpu.VMEM((2, S * DBLK, C), jnp.float32),
            pltpu.SemaphoreType.DMA,
            pltpu.SemaphoreType.DMA,
        ],
    )
    def _body(xt, yt, cdf_hbm, rq_hbm, out_x, out_y,
              cdf_v, rq_v, idx_v, in_v, out_v, isem, osem):
        wid = lax.axis_index("s") * NC + lax.axis_index("c")
        pltpu.sync_copy(cdf_hbm, cdf_v)
        pltpu.sync_copy(rq_hbm, rq_v)

        def search(q, _):
            rq = rq_v[pl.ds(q * 16, 16)]
            lo = jnp.zeros((16,), jnp.int32)
            hi = jnp.full((16,), C, jnp.int32)
            for _step in range(STEPS):
                mid = (lo + hi) >> 1
                pred = plsc.load_gather(cdf_v, [mid]) < rq
                lo = jnp.where(pred, mid + 1, lo)
                hi = jnp.where(pred, hi, mid)
            idx_v[pl.ds(q * 16, 16)] = lo
            return _

        lax.fori_loop(0, S * C // 16, search, None)

        tbase = wid * D
        obase = wid * S * D

        def permute_block(buf, _):
            def qstep(q, _):
                for s in range(S):
                    idx = idx_v[pl.ds(s * C + q * 16, 16)]
                    for dd in range(DBLK):
                        v = plsc.load_gather(
                            in_v, [jnp.full((16,), buf, jnp.int32),
                                   jnp.full((16,), dd, jnp.int32), idx])
                        out_v[buf, s * DBLK + dd, pl.ds(q * 16, 16)] = v
                return _
            lax.fori_loop(0, QV, qstep, None)

        tasks = [(xt, out_x), (yt, out_y)]
        wr = [[None] * S, [None] * S]
        ld = [None, None]
        ld[0] = pltpu.async_copy(
            tasks[0][0].at[pl.ds(tbase, DBLK)], in_v.at[0], isem)
        for t, (table, out) in enumerate(tasks):
            for blk in range(NBLK):
                buf = (t * NBLK + blk) & 1
                nt, nblk = (t, blk + 1) if blk + 1 < NBLK else (t + 1, 0)
                if nt < 2:
                    ld[1 - buf] = pltpu.async_copy(
                        tasks[nt][0].at[pl.ds(tbase + nblk * DBLK, DBLK)],
                        in_v.at[1 - buf], isem)
                ld[buf].wait()
                for s in range(S):
                    if wr[buf][s] is not None:
                        wr[buf][s].wait()
                permute_block(buf, None)
                for s in range(S):
                    wr[buf][s] = pltpu.async_copy(
                        out_v.at[buf, pl.ds(s * DBLK, DBLK)],
                        out.at[pl.ds(obase + s * D + blk * DBLK, DBLK)],
                        osem)
        for bufw in wr:
            for w in bufw:
                if w is not None:
                    w.wait()

    return _body


def kernel(x_ctx, y_ctx, mask_ctx, num_samples):
    key = jax.random.key(42)
    cdf = jnp.cumsum(mask_ctx[0])
    u = jax.random.uniform(key, (C, S), dtype=cdf.dtype)
    rq = (cdf[-1] * (1 - u)).T.reshape(-1)

    xt = jnp.swapaxes(x_ctx, 1, 2).reshape(B * D, C)
    yt = jnp.swapaxes(y_ctx, 1, 2).reshape(B * D, C)
    out_x, out_y = _make_sc_kernel()(xt, yt, cdf, rq)
    out_x = jnp.swapaxes(out_x.reshape(B, S, D, C), 2, 3)
    out_y = jnp.swapaxes(out_y.reshape(B, S, D, C), 2, 3)
    return (out_x, out_y)


# speedup vs baseline: 1.0030x; 1.0030x over previous
import functools

import jax
import jax.numpy as jnp
from jax import lax
from jax.experimental import pallas as pl
from jax.experimental.pallas import tpu as pltpu
from jax.experimental.pallas import tpu_sc as plsc

B, C, D, S = 32, 2048, 64, 4
NC, NS = 2, 16
NW = NC * NS
DBLK = 4
NBLK = D // DBLK
QV = C // 16
STEPS = C.bit_length()


@functools.cache
def _make_sc_kernel():
    @functools.partial(
        pl.kernel,
        out_type=(jax.ShapeDtypeStruct((B * S * D, C), jnp.float32),
                  jax.ShapeDtypeStruct((B * S * D, C), jnp.float32)),
        mesh=plsc.VectorSubcoreMesh(core_axis_name="c", subcore_axis_name="s"),
        compiler_params=pltpu.CompilerParams(use_tc_tiling_on_sc=False,
                                             needs_layout_passes=False),
        scratch_types=[
            pltpu.VMEM((C,), jnp.float32),
            pltpu.VMEM((S * C,), jnp.float32),
            pltpu.VMEM((S * C,), jnp.int32),
            pltpu.VMEM((2, DBLK, C), jnp.float32),
            pltpu.VMEM((2, S * DBLK, C), jnp.float32),
            pltpu.SemaphoreType.DMA,
            pltpu.SemaphoreType.DMA,
        ],
    )
    def _body(xt, yt, cdf_hbm, rq_hbm, out_x, out_y,
              cdf_v, rq_v, idx_v, in_v, out_v, isem, osem):
        wid = lax.axis_index("s") * NC + lax.axis_index("c")
        pltpu.sync_copy(cdf_hbm, cdf_v)
        pltpu.sync_copy(rq_hbm, rq_v)

        def search(q, _):
            rq = rq_v[pl.ds(q * 16, 16)]
            lo = jnp.zeros((16,), jnp.int32)
            hi = jnp.full((16,), C, jnp.int32)
            for _step in range(STEPS):
                mid = (lo + hi) >> 1
                pred = plsc.load_gather(cdf_v, [mid]) < rq
                lo = jnp.where(pred, mid + 1, lo)
                hi = jnp.where(pred, hi, mid)
            idx_v[pl.ds(q * 16, 16)] = lo
            return _

        lax.fori_loop(0, S * C // 16, search, None)

        tbase = wid * D
        obase = wid * S * D

        def permute_block(buf, _):
            def qstep(q, _):
                for s in range(S):
                    idx = idx_v[pl.ds(s * C + q * 16, 16)]
                    for dd in range(DBLK):
                        v = plsc.load_gather(in_v.at[buf, dd], [idx])
                        out_v[buf, s * DBLK + dd, pl.ds(q * 16, 16)] = v
                return _
            lax.fori_loop(0, QV, qstep, None)

        tasks = [(xt, out_x), (yt, out_y)]
        wr = [[None] * S, [None] * S]
        ld = [None, None]
        ld[0] = pltpu.async_copy(
            tasks[0][0].at[pl.ds(tbase, DBLK)], in_v.at[0], isem)
        for t, (table, out) in enumerate(tasks):
            for blk in range(NBLK):
                buf = (t * NBLK + blk) & 1
                nt, nblk = (t, blk + 1) if blk + 1 < NBLK else (t + 1, 0)
                if nt < 2:
                    ld[1 - buf] = pltpu.async_copy(
                        tasks[nt][0].at[pl.ds(tbase + nblk * DBLK, DBLK)],
                        in_v.at[1 - buf], isem)
                ld[buf].wait()
                for s in range(S):
                    if wr[buf][s] is not None:
                        wr[buf][s].wait()
                permute_block(buf, None)
                for s in range(S):
                    wr[buf][s] = pltpu.async_copy(
                        out_v.at[buf, pl.ds(s * DBLK, DBLK)],
                        out.at[pl.ds(obase + s * D + blk * DBLK, DBLK)],
                        osem)
        for bufw in wr:
            for w in bufw:
                if w is not None:
                    w.wait()

    return _body


def kernel(x_ctx, y_ctx, mask_ctx, num_samples):
    key = jax.random.key(42)
    cdf = jnp.cumsum(mask_ctx[0])
    u = jax.random.uniform(key, (C, S), dtype=cdf.dtype)
    rq = (cdf[-1] * (1 - u)).T.reshape(-1)

    xt = jnp.swapaxes(x_ctx, 1, 2).reshape(B * D, C)
    yt = jnp.swapaxes(y_ctx, 1, 2).reshape(B * D, C)
    out_x, out_y = _make_sc_kernel()(xt, yt, cdf, rq)
    out_x = jnp.swapaxes(out_x.reshape(B, S, D, C), 2, 3)
    out_y = jnp.swapaxes(out_y.reshape(B, S, D, C), 2, 3)
    return (out_x, out_y)


# speedup vs baseline: 1.1884x; 1.1848x over previous
import functools

import jax
import jax.numpy as jnp
from jax import lax
from jax.experimental import pallas as pl
from jax.experimental.pallas import tpu as pltpu
from jax.experimental.pallas import tpu_sc as plsc

B, C, D, S = 32, 2048, 64, 4
R = B * S * C
NC, NS = 2, 16
NW = NC * NS
ROWS_W = R // NW
CHUNK = 256
NCHUNK = ROWS_W // CHUNK
IDXC = 128
STEPS = C.bit_length()


@functools.cache
def _make_sc_kernel():
    @functools.partial(
        pl.kernel,
        out_type=(jax.ShapeDtypeStruct((R, D), jnp.float32),
                  jax.ShapeDtypeStruct((R, D), jnp.float32)),
        mesh=plsc.VectorSubcoreMesh(core_axis_name="c", subcore_axis_name="s"),
        compiler_params=pltpu.CompilerParams(use_tc_tiling_on_sc=False,
                                             needs_layout_passes=False),
        scratch_types=[
            pltpu.VMEM((C,), jnp.float32),
            pltpu.VMEM((ROWS_W,), jnp.float32),
            pltpu.VMEM((ROWS_W,), jnp.int32),
            pltpu.VMEM((2, CHUNK, D), jnp.float32),
            pltpu.VMEM((2, CHUNK, D), jnp.float32),
            pltpu.SemaphoreType.DMA,
            pltpu.SemaphoreType.DMA,
            pltpu.SemaphoreType.DMA,
        ],
    )
    def _body(xf, yf, cdf_hbm, rq_hbm, out_x, out_y,
              cdf_v, rq_v, idx_v, xrow_v, yrow_v, gsem, wsx, wsy):
        wid = lax.axis_index("s") * NC + lax.axis_index("c")
        base = wid * ROWS_W
        pltpu.sync_copy(cdf_hbm, cdf_v)
        pltpu.sync_copy(rq_hbm, rq_v)

        boff = jnp.full((16,), wid * C, dtype=jnp.int32)

        def search(q, _):
            rq = rq_v[pl.ds(q * 16, 16)]
            lo = jnp.zeros((16,), jnp.int32)
            hi = jnp.full((16,), C, jnp.int32)
            for _step in range(STEPS):
                mid = (lo + hi) >> 1
                pred = plsc.load_gather(cdf_v, [mid]) < rq
                lo = jnp.where(pred, mid + 1, lo)
                hi = jnp.where(pred, hi, mid)
            idx_v[pl.ds(q * 16, 16)] = lo + boff
            return _

        def gather_chunk(table, j, row_v, buf):
            return [pltpu.async_copy(
                        table.at[idx_v.at[pl.ds(j * CHUNK + k * IDXC, IDXC)]],
                        row_v.at[buf, pl.ds(k * IDXC, IDXC)], gsem)
                    for k in range(CHUNK // IDXC)]

        QPC = CHUNK // 16
        lax.fori_loop(0, QPC, search, None)
        wx = [None, None]
        wy = [None, None]
        for j in range(NCHUNK):
            buf = j & 1
            if wx[buf] is not None:
                wx[buf].wait()
            if wy[buf] is not None:
                wy[buf].wait()
            gx = gather_chunk(xf, j, xrow_v, buf)
            gy = gather_chunk(yf, j, yrow_v, buf)
            if j + 1 < NCHUNK:
                lax.fori_loop((j + 1) * QPC, (j + 2) * QPC, search, None)
            for cp in gx:
                cp.wait()
            wx[buf] = pltpu.async_copy(
                xrow_v.at[buf], out_x.at[pl.ds(base + j * CHUNK, CHUNK)], wsx)
            for cp in gy:
                cp.wait()
            wy[buf] = pltpu.async_copy(
                yrow_v.at[buf], out_y.at[pl.ds(base + j * CHUNK, CHUNK)], wsy)
        wx[0].wait()
        wx[1].wait()
        wy[0].wait()
        wy[1].wait()

    return _body


def kernel(x_ctx, y_ctx, mask_ctx, num_samples):
    key = jax.random.key(42)
    cdf = jnp.cumsum(mask_ctx[0])
    u = jax.random.uniform(key, (C, S), dtype=cdf.dtype)
    rq = (cdf[-1] * (1 - u)).T.reshape(-1)

    out_x, out_y = _make_sc_kernel()(
        x_ctx.reshape(B * C, D), y_ctx.reshape(B * C, D), cdf, rq)
    return (out_x.reshape(B, S, C, D), out_y.reshape(B, S, C, D))
